# baseline (speedup 1.0000x reference)
import functools

import jax
import jax.numpy as jnp
from jax import lax
from jax.experimental import pallas as pl
from jax.experimental.pallas import tpu as pltpu
from jax.experimental.pallas import tpu_sc as plsc

NC, NS = 2, 16
NW = NC * NS
GW = 128
NG = 4
CHUNK = GW * NG
LANES = 16


def _sc_embed(x3, token_table, pos_table, *, batch, embed, maxlen):
    mesh = plsc.VectorSubcoreMesh(core_axis_name="c", subcore_axis_name="s")
    egroups = CHUNK // LANES

    @functools.partial(
        pl.kernel,
        out_type=jax.ShapeDtypeStruct((maxlen * embed, batch), jnp.float32),
        mesh=mesh,
        scratch_types=[
            pltpu.VMEM((2, NG, GW), jnp.int32),
            pltpu.VMEM((2, CHUNK, embed), jnp.float32),
            pltpu.VMEM((2, embed, CHUNK), jnp.float32),
            pltpu.VMEM((CHUNK,), jnp.float32),
            pltpu.VMEM((maxlen + 1, embed), jnp.float32),
            pltpu.SemaphoreType.DMA,
            pltpu.SemaphoreType.DMA,
            pltpu.SemaphoreType.DMA,
            pltpu.SemaphoreType.DMA,
            pltpu.SemaphoreType.DMA,
            pltpu.SemaphoreType.DMA,
            pltpu.SemaphoreType.DMA,
        ],
        compiler_params=pltpu.CompilerParams(use_tc_tiling_on_sc=False,
                                             needs_layout_passes=False),
    )
    def sc_kernel(x_hbm, tok_hbm, pos_hbm, out_hbm,
                  idx_v, tok_v, out_v, mf_v, pos_v,
                  sp, si0, si1, sg0, sg1, so0, so1):
        wid = lax.axis_index("s") * NC + lax.axis_index("c")
        b0 = wid * CHUNK
        brow = wid * NG

        pltpu.async_copy(pos_hbm, pos_v, sp).wait()

        def fire_idx(slot, l, sem):
            pltpu.async_copy(x_hbm.at[l, pl.ds(brow, NG)], idx_v.at[slot], sem)

        def wait_idx(slot, sem):
            pltpu.make_async_copy(
                x_hbm.at[0, pl.ds(0, NG)], idx_v.at[slot], sem).wait()

        def wait_out(slot, sem):
            pltpu.make_async_copy(
                out_v.at[slot],
                out_hbm.at[pl.ds(0, embed), pl.ds(b0, CHUNK)], sem).wait()

        def fire_gathers(slot, sem_g):
            for k in range(NG):
                pltpu.async_copy(tok_hbm.at[idx_v.at[slot, k]],
                                 tok_v.at[slot, pl.ds(k * GW, GW)], sem_g)

        def drain_gathers(slot, sem_g):
            for k in range(NG):
                pltpu.make_async_copy(
                    tok_hbm.at[pl.ds(0, GW)],
                    tok_v.at[slot, pl.ds(k * GW, GW)], sem_g).wait()

        def splat(vec, lane):
            idx = jnp.broadcast_to(lane.astype(jnp.int32), (LANES,))
            return lax.gather(
                vec, idx[:, None],
                dimension_numbers=lax.GatherDimensionNumbers(
                    offset_dims=(), collapsed_slice_dims=(0,),
                    start_index_map=(0,)),
                slice_sizes=(1,),
                mode=lax.GatherScatterMode.PROMISE_IN_BOUNDS)

        def transpose_add(slot, l):
            mn = idx_v[slot, 0, pl.ds(0, LANES)]
            for k in range(NG):
                for j in range(GW // LANES):
                    if k == 0 and j == 0:
                        continue
                    mn = jnp.minimum(mn, idx_v[slot, k, pl.ds(j * LANES, LANES)])
            has_zero = lax.reduce_min(mn, (0,)) == 0
            iota32 = lax.iota(jnp.int32, LANES)

            @pl.when(jnp.logical_not(has_zero))
            def _():
                @pl.loop(0, embed)
                def _(e):
                    h16 = (e // LANES) * LANES
                    row1 = pos_v[l + 1, pl.ds(h16, LANES)]
                    p1 = splat(row1, e - h16)
                    ecol = jnp.broadcast_to(e, (LANES,))
                    for g in range(egroups):
                        rows = g * LANES + iota32
                        tokt = plsc.load_gather(tok_v.at[slot], [rows, ecol])
                        out_v[slot, e, pl.ds(g * LANES, LANES)] = tokt + p1

            @pl.when(has_zero)
            def _():
                for g in range(egroups):
                    k, j = divmod(g, GW // LANES)
                    xg = idx_v[slot, k, pl.ds(j * LANES, LANES)]
                    mf_v[pl.ds(g * LANES, LANES)] = jnp.where(
                        xg == 0, jnp.float32(1.0), jnp.float32(0.0))

                @pl.loop(0, embed)
                def _(e):
                    h16 = (e // LANES) * LANES
                    row1 = pos_v[l + 1, pl.ds(h16, LANES)]
                    row0 = pos_v[0, pl.ds(h16, LANES)]
                    p1 = splat(row1, e - h16)
                    d = splat(row0, e - h16) - p1
                    ecol = jnp.broadcast_to(e, (LANES,))
                    for g in range(egroups):
                        rows = g * LANES + iota32
                        tokt = plsc.load_gather(tok_v.at[slot], [rows, ecol])
                        mf = mf_v[pl.ds(g * LANES, LANES)]
                        out_v[slot, e, pl.ds(g * LANES, LANES)] = (
                            tokt + p1 + mf * d)

        def fire_out(slot, l, sem_o):
            pltpu.async_copy(
                out_v.at[slot],
                out_hbm.at[pl.ds(l * embed, embed), pl.ds(b0, CHUNK)], sem_o)

        def phase_fire(slot, l, sem_i, sem_o):
            wait_idx(slot, sem_i)

            @pl.when(l >= 2)
            def _():
                wait_out(slot, sem_o)

            fire_gathers(slot, (sg0, sg1)[slot])

        def phase_drain(slot, l, sem_i, sem_g, sem_o):
            drain_gathers(slot, sem_g)

            @pl.when(l + 2 < maxlen)
            def _():
                fire_idx(slot, l + 2, sem_i)

            transpose_add(slot, l)
            fire_out(slot, l, sem_o)

        fire_idx(0, 0, si0)
        fire_idx(1, 1, si1)

        @pl.loop(0, maxlen, step=2)
        def _(l0):
            phase_fire(0, l0, si0, so0)
            phase_fire(1, l0 + 1, si1, so1)
            phase_drain(0, l0, si0, sg0, so0)
            phase_drain(1, l0 + 1, si1, sg1, so1)

        wait_out(0, so0)
        wait_out(1, so1)

    return sc_kernel(x3, token_table, pos_table)


def kernel(x, token_table, pos_table):
    batch, maxlen = x.shape
    embed = token_table.shape[1]
    x3 = x.T.reshape(maxlen, batch // GW, GW).astype(jnp.int32)
    out2d = _sc_embed(x3, token_table, pos_table,
                      batch=batch, embed=embed, maxlen=maxlen)
    return out2d.reshape(maxlen, embed, batch).transpose(2, 0, 1)


# speedup vs baseline: 1.2369x; 1.2369x over previous
import functools

import jax
import jax.numpy as jnp
from jax import lax
from jax.experimental import pallas as pl
from jax.experimental.pallas import tpu as pltpu
from jax.experimental.pallas import tpu_sc as plsc

NC, NS = 2, 16
NW = NC * NS
GW = 128
NG = 4
CHUNK = GW * NG
LANES = 16


def _sc_embed(x3, token_table, pos_table, *, batch, embed, maxlen):
    mesh = plsc.VectorSubcoreMesh(core_axis_name="c", subcore_axis_name="s")
    egroups = CHUNK // LANES

    @functools.partial(
        pl.kernel,
        out_type=jax.ShapeDtypeStruct((maxlen * embed, batch), jnp.float32),
        mesh=mesh,
        scratch_types=[
            pltpu.VMEM((2, NG, GW), jnp.int32),
            pltpu.VMEM((2, CHUNK, embed), jnp.float32),
            pltpu.VMEM((2, embed, CHUNK), jnp.float32),
            pltpu.VMEM((CHUNK,), jnp.float32),
            pltpu.VMEM((maxlen + 1, embed), jnp.float32),
            pltpu.SemaphoreType.DMA,
            pltpu.SemaphoreType.DMA,
            pltpu.SemaphoreType.DMA,
            pltpu.SemaphoreType.DMA,
            pltpu.SemaphoreType.DMA,
            pltpu.SemaphoreType.DMA,
            pltpu.SemaphoreType.DMA,
        ],
        compiler_params=pltpu.CompilerParams(use_tc_tiling_on_sc=False,
                                             needs_layout_passes=False),
    )
    def sc_kernel(x_hbm, tok_hbm, pos_hbm, out_hbm,
                  idx_v, tok_v, out_v, mf_v, pos_v,
                  sp, si0, si1, sg0, sg1, so0, so1):
        wid = lax.axis_index("s") * NC + lax.axis_index("c")
        b0 = wid * CHUNK
        brow = wid * NG

        pltpu.async_copy(pos_hbm, pos_v, sp).wait()

        def fire_idx(slot, l, sem):
            pltpu.async_copy(x_hbm.at[l, pl.ds(brow, NG)], idx_v.at[slot], sem)

        def wait_idx(slot, sem):
            pltpu.make_async_copy(
                x_hbm.at[0, pl.ds(0, NG)], idx_v.at[slot], sem).wait()

        def wait_out(slot, sem):
            pltpu.make_async_copy(
                out_v.at[slot],
                out_hbm.at[pl.ds(0, embed), pl.ds(b0, CHUNK)], sem).wait()

        def fire_gathers(slot, sem_g):
            for k in range(NG):
                pltpu.async_copy(tok_hbm.at[idx_v.at[slot, k]],
                                 tok_v.at[slot, pl.ds(k * GW, GW)], sem_g)

        def drain_gathers(slot, sem_g):
            for k in range(NG):
                pltpu.make_async_copy(
                    tok_hbm.at[pl.ds(0, GW)],
                    tok_v.at[slot, pl.ds(k * GW, GW)], sem_g).wait()

        def splat(vec, lane):
            idx = jnp.broadcast_to(lane.astype(jnp.int32), (LANES,))
            return lax.gather(
                vec, idx[:, None],
                dimension_numbers=lax.GatherDimensionNumbers(
                    offset_dims=(), collapsed_slice_dims=(0,),
                    start_index_map=(0,)),
                slice_sizes=(1,),
                mode=lax.GatherScatterMode.PROMISE_IN_BOUNDS)

        def transpose_add(slot, l):
            mn = idx_v[slot, 0, pl.ds(0, LANES)]
            for k in range(NG):
                for j in range(GW // LANES):
                    if k == 0 and j == 0:
                        continue
                    mn = jnp.minimum(mn, idx_v[slot, k, pl.ds(j * LANES, LANES)])
            has_zero = lax.reduce_min(mn, (0,)) == 0
            iota32 = lax.iota(jnp.int32, LANES)

            @pl.when(jnp.logical_not(has_zero))
            def _():
                @pl.loop(0, embed)
                def _(e):
                    h16 = (e // LANES) * LANES
                    row1 = pos_v[l + 1, pl.ds(h16, LANES)]
                    p1 = splat(row1, e - h16)
                    ecol = jnp.broadcast_to(e, (LANES,))

                    @plsc.parallel_loop(0, egroups, unroll=8)
                    def _(g):
                        rows = g * LANES + iota32
                        tokt = plsc.load_gather(tok_v.at[slot], [rows, ecol])
                        out_v[slot, e, pl.ds(g * LANES, LANES)] = tokt + p1

            @pl.when(has_zero)
            def _():
                for g in range(egroups):
                    k, j = divmod(g, GW // LANES)
                    xg = idx_v[slot, k, pl.ds(j * LANES, LANES)]
                    mf_v[pl.ds(g * LANES, LANES)] = jnp.where(
                        xg == 0, jnp.float32(1.0), jnp.float32(0.0))

                @pl.loop(0, embed)
                def _(e):
                    h16 = (e // LANES) * LANES
                    row1 = pos_v[l + 1, pl.ds(h16, LANES)]
                    row0 = pos_v[0, pl.ds(h16, LANES)]
                    p1 = splat(row1, e - h16)
                    d = splat(row0, e - h16) - p1
                    ecol = jnp.broadcast_to(e, (LANES,))

                    @plsc.parallel_loop(0, egroups, unroll=8)
                    def _(g):
                        rows = g * LANES + iota32
                        tokt = plsc.load_gather(tok_v.at[slot], [rows, ecol])
                        mf = mf_v[pl.ds(g * LANES, LANES)]
                        out_v[slot, e, pl.ds(g * LANES, LANES)] = (
                            tokt + p1 + mf * d)

        def fire_out(slot, l, sem_o):
            pltpu.async_copy(
                out_v.at[slot],
                out_hbm.at[pl.ds(l * embed, embed), pl.ds(b0, CHUNK)], sem_o)

        def phase_fire(slot, l, sem_i, sem_o):
            wait_idx(slot, sem_i)

            @pl.when(l >= 2)
            def _():
                wait_out(slot, sem_o)

            fire_gathers(slot, (sg0, sg1)[slot])

        def phase_drain(slot, l, sem_i, sem_g, sem_o):
            drain_gathers(slot, sem_g)

            @pl.when(l + 2 < maxlen)
            def _():
                fire_idx(slot, l + 2, sem_i)

            transpose_add(slot, l)
            fire_out(slot, l, sem_o)

        fire_idx(0, 0, si0)
        fire_idx(1, 1, si1)

        @pl.loop(0, maxlen, step=2)
        def _(l0):
            phase_fire(0, l0, si0, so0)
            phase_fire(1, l0 + 1, si1, so1)
            phase_drain(0, l0, si0, sg0, so0)
            phase_drain(1, l0 + 1, si1, sg1, so1)

        wait_out(0, so0)
        wait_out(1, so1)

    return sc_kernel(x3, token_table, pos_table)


def kernel(x, token_table, pos_table):
    batch, maxlen = x.shape
    embed = token_table.shape[1]
    x3 = x.T.reshape(maxlen, batch // GW, GW).astype(jnp.int32)
    out2d = _sc_embed(x3, token_table, pos_table,
                      batch=batch, embed=embed, maxlen=maxlen)
    return out2d.reshape(maxlen, embed, batch).transpose(2, 0, 1)


# speedup vs baseline: 2.8563x; 2.3093x over previous
import functools

import jax
import jax.numpy as jnp
from jax import lax
from jax.experimental import pallas as pl
from jax.experimental.pallas import tpu as pltpu
from jax.experimental.pallas import tpu_sc as plsc

NC, NS = 2, 16
NW = NC * NS
GW = 128
NG = 4
CHUNK = GW * NG
LANES = 16


def _sc_embed(x3, token_table, pos_table, *, batch, embed, maxlen):
    mesh = plsc.VectorSubcoreMesh(core_axis_name="c", subcore_axis_name="s")
    egroups = CHUNK // LANES

    @functools.partial(
        pl.kernel,
        out_type=jax.ShapeDtypeStruct((maxlen * embed, batch), jnp.float32),
        mesh=mesh,
        scratch_types=[
            pltpu.VMEM((2, NG, GW), jnp.int32),
            pltpu.VMEM((2, CHUNK, embed), jnp.float32),
            pltpu.VMEM((2, embed, CHUNK + 1), jnp.float32),
            pltpu.VMEM((maxlen + 1, embed), jnp.float32),
            pltpu.SemaphoreType.DMA,
            pltpu.SemaphoreType.DMA,
            pltpu.SemaphoreType.DMA,
            pltpu.SemaphoreType.DMA,
            pltpu.SemaphoreType.DMA,
            pltpu.SemaphoreType.DMA,
            pltpu.SemaphoreType.DMA,
        ],
        compiler_params=pltpu.CompilerParams(use_tc_tiling_on_sc=False,
                                             needs_layout_passes=False),
    )
    def sc_kernel(x_hbm, tok_hbm, pos_hbm, out_hbm,
                  idx_v, tok_v, out_v, pos_v,
                  sp, si0, si1, sg0, sg1, so0, so1):
        wid = lax.axis_index("s") * NC + lax.axis_index("c")
        b0 = wid * CHUNK
        brow = wid * NG

        pltpu.async_copy(pos_hbm, pos_v, sp).wait()

        def fire_idx(slot, l, sem):
            pltpu.async_copy(x_hbm.at[l, pl.ds(brow, NG)], idx_v.at[slot], sem)

        def wait_idx(slot, sem):
            pltpu.make_async_copy(
                x_hbm.at[0, pl.ds(0, NG)], idx_v.at[slot], sem).wait()

        def wait_out(slot, sem):
            pltpu.make_async_copy(
                out_v.at[slot, pl.ds(0, embed), pl.ds(0, CHUNK)],
                out_hbm.at[pl.ds(0, embed), pl.ds(b0, CHUNK)], sem).wait()

        def fire_gathers(slot, sem_g):
            for k in range(NG):
                pltpu.async_copy(tok_hbm.at[idx_v.at[slot, k]],
                                 tok_v.at[slot, pl.ds(k * GW, GW)], sem_g)

        def drain_gathers(slot, sem_g):
            for k in range(NG):
                pltpu.make_async_copy(
                    tok_hbm.at[pl.ds(0, GW)],
                    tok_v.at[slot, pl.ds(k * GW, GW)], sem_g).wait()

        def splat(vec, lane):
            idx = jnp.broadcast_to(lane.astype(jnp.int32), (LANES,))
            return lax.gather(
                vec, idx[:, None],
                dimension_numbers=lax.GatherDimensionNumbers(
                    offset_dims=(), collapsed_slice_dims=(0,),
                    start_index_map=(0,)),
                slice_sizes=(1,),
                mode=lax.GatherScatterMode.PROMISE_IN_BOUNDS)

        def transpose_add(slot, l):
            iota32 = lax.iota(jnp.int32, LANES)
            p1h = [pos_v[l + 1, pl.ds(h * LANES, LANES)]
                   for h in range(embed // LANES)]
            ecols = [h * LANES + iota32 for h in range(embed // LANES)]

            @plsc.parallel_loop(0, CHUNK, unroll=8)
            def _(r):
                rv = jnp.broadcast_to(r, (LANES,))
                for h in range(embed // LANES):
                    tok = tok_v[slot, r, pl.ds(h * LANES, LANES)]
                    plsc.store_scatter(out_v.at[slot], [ecols[h], rv],
                                       tok + p1h[h])

            mn = idx_v[slot, 0, pl.ds(0, LANES)]
            for k in range(NG):
                for j in range(GW // LANES):
                    if k == 0 and j == 0:
                        continue
                    mn = jnp.minimum(mn, idx_v[slot, k, pl.ds(j * LANES, LANES)])
            has_zero = lax.reduce_min(mn, (0,)) == 0

            @pl.when(has_zero)
            def _():
                p0h = [pos_v[0, pl.ds(h * LANES, LANES)]
                       for h in range(embed // LANES)]

                @pl.loop(0, egroups)
                def _(g):
                    xg = idx_v[slot, g // (GW // LANES),
                               pl.ds((g % (GW // LANES)) * LANES, LANES)]
                    m = xg == 0
                    rows = g * LANES + iota32

                    @pl.when(jnp.any(m))
                    def _():
                        for h in range(embed // LANES):
                            for j in range(LANES):
                                e = h * LANES + j
                                ecol = jnp.broadcast_to(
                                    jnp.int32(e), (LANES,))
                                tokt = plsc.load_gather(
                                    tok_v.at[slot], [rows, ecol], mask=m)
                                p0 = splat(p0h[h], jnp.int32(j))
                                plsc.store_scatter(
                                    out_v.at[slot], [ecol, rows],
                                    tokt + p0, mask=m)

        def fire_out(slot, l, sem_o):
            pltpu.async_copy(
                out_v.at[slot, pl.ds(0, embed), pl.ds(0, CHUNK)],
                out_hbm.at[pl.ds(l * embed, embed), pl.ds(b0, CHUNK)], sem_o)

        def phase_fire(slot, l, sem_i, sem_o):
            wait_idx(slot, sem_i)

            @pl.when(l >= 2)
            def _():
                wait_out(slot, sem_o)

            fire_gathers(slot, (sg0, sg1)[slot])

        def phase_drain(slot, l, sem_i, sem_g, sem_o):
            drain_gathers(slot, sem_g)
            transpose_add(slot, l)

            @pl.when(l + 2 < maxlen)
            def _():
                fire_idx(slot, l + 2, sem_i)

            fire_out(slot, l, sem_o)

        fire_idx(0, 0, si0)
        fire_idx(1, 1, si1)

        @pl.loop(0, maxlen, step=2)
        def _(l0):
            phase_fire(0, l0, si0, so0)
            phase_fire(1, l0 + 1, si1, so1)
            phase_drain(0, l0, si0, sg0, so0)
            phase_drain(1, l0 + 1, si1, sg1, so1)

        wait_out(0, so0)
        wait_out(1, so1)

    return sc_kernel(x3, token_table, pos_table)


def kernel(x, token_table, pos_table):
    batch, maxlen = x.shape
    embed = token_table.shape[1]
    x3 = x.T.reshape(maxlen, batch // GW, GW).astype(jnp.int32)
    out2d = _sc_embed(x3, token_table, pos_table,
                      batch=batch, embed=embed, maxlen=maxlen)
    return out2d.reshape(maxlen, embed, batch).transpose(2, 0, 1)


# speedup vs baseline: 3.9435x; 1.3807x over previous
import functools

import jax
import jax.numpy as jnp
from jax import lax
from jax.experimental import pallas as pl
from jax.experimental.pallas import tpu as pltpu
from jax.experimental.pallas import tpu_sc as plsc

NC, NS = 2, 16
NW = NC * NS
GW = 128
NG = 4
CHUNK = GW * NG
LANES = 16


def _sc_embed(x3, token_table, pos_table, *, batch, embed, maxlen):
    mesh = plsc.VectorSubcoreMesh(core_axis_name="c", subcore_axis_name="s")
    egroups = CHUNK // LANES

    @functools.partial(
        pl.kernel,
        out_type=jax.ShapeDtypeStruct(
            (maxlen, embed // 8, batch // 128, 8, 128), jnp.float32),
        mesh=mesh,
        scratch_types=[
            pltpu.VMEM((2, NG, GW), jnp.int32),
            pltpu.VMEM((2, CHUNK, embed), jnp.float32),
            pltpu.VMEM((2, embed // 8, 8, CHUNK + 1), jnp.float32),
            pltpu.VMEM((maxlen + 1, embed), jnp.float32),
            pltpu.SemaphoreType.DMA,
            pltpu.SemaphoreType.DMA,
            pltpu.SemaphoreType.DMA,
            pltpu.SemaphoreType.DMA,
            pltpu.SemaphoreType.DMA,
            pltpu.SemaphoreType.DMA,
            pltpu.SemaphoreType.DMA,
        ],
        compiler_params=pltpu.CompilerParams(use_tc_tiling_on_sc=False,
                                             needs_layout_passes=False),
    )
    def sc_kernel(x_hbm, tok_hbm, pos_hbm, out_hbm,
                  idx_v, tok_v, out_v, pos_v,
                  sp, si0, si1, sg0, sg1, so0, so1):
        wid = lax.axis_index("s") * NC + lax.axis_index("c")
        b0 = wid * CHUNK
        brow = wid * NG

        pltpu.async_copy(pos_hbm, pos_v, sp).wait()

        def fire_idx(slot, l, sem):
            pltpu.async_copy(x_hbm.at[l, pl.ds(brow, NG)], idx_v.at[slot], sem)

        def wait_idx(slot, sem):
            pltpu.make_async_copy(
                x_hbm.at[0, pl.ds(0, NG)], idx_v.at[slot], sem).wait()

        def wait_out(slot, sem):
            for bb in range(CHUNK // 128):
                pltpu.make_async_copy(
                    out_v.at[slot, pl.ds(0, embed // 8), pl.ds(0, 8),
                             pl.ds(bb * 128, 128)],
                    out_hbm.at[0, pl.ds(0, embed // 8), 0,
                               pl.ds(0, 8), pl.ds(0, 128)], sem).wait()

        def fire_gathers(slot, sem_g):
            for k in range(NG):
                pltpu.async_copy(tok_hbm.at[idx_v.at[slot, k]],
                                 tok_v.at[slot, pl.ds(k * GW, GW)], sem_g)

        def drain_gathers(slot, sem_g):
            for k in range(NG):
                pltpu.make_async_copy(
                    tok_hbm.at[pl.ds(0, GW)],
                    tok_v.at[slot, pl.ds(k * GW, GW)], sem_g).wait()

        def splat(vec, lane):
            idx = jnp.broadcast_to(lane.astype(jnp.int32), (LANES,))
            return lax.gather(
                vec, idx[:, None],
                dimension_numbers=lax.GatherDimensionNumbers(
                    offset_dims=(), collapsed_slice_dims=(0,),
                    start_index_map=(0,)),
                slice_sizes=(1,),
                mode=lax.GatherScatterMode.PROMISE_IN_BOUNDS)

        def transpose_add(slot, l):
            iota32 = lax.iota(jnp.int32, LANES)
            p1h = [pos_v[l + 1, pl.ds(h * LANES, LANES)]
                   for h in range(embed // LANES)]
            e8cols = [(h * LANES + iota32) // 8 for h in range(embed // LANES)]
            eincols = [(h * LANES + iota32) % 8 for h in range(embed // LANES)]

            @plsc.parallel_loop(0, CHUNK, unroll=8)
            def _(r):
                rv = jnp.broadcast_to(r, (LANES,))
                for h in range(embed // LANES):
                    tok = tok_v[slot, r, pl.ds(h * LANES, LANES)]
                    plsc.store_scatter(out_v.at[slot],
                                       [e8cols[h], eincols[h], rv],
                                       tok + p1h[h])

            mn = idx_v[slot, 0, pl.ds(0, LANES)]
            for k in range(NG):
                for j in range(GW // LANES):
                    if k == 0 and j == 0:
                        continue
                    mn = jnp.minimum(mn, idx_v[slot, k, pl.ds(j * LANES, LANES)])
            has_zero = lax.reduce_min(mn, (0,)) == 0

            @pl.when(has_zero)
            def _():
                p0h = [pos_v[0, pl.ds(h * LANES, LANES)]
                       for h in range(embed // LANES)]

                @pl.loop(0, egroups)
                def _(g):
                    xg = idx_v[slot, g // (GW // LANES),
                               pl.ds((g % (GW // LANES)) * LANES, LANES)]
                    m = xg == 0
                    rows = g * LANES + iota32

                    @pl.when(jnp.any(m))
                    def _():
                        for h in range(embed // LANES):
                            for j in range(LANES):
                                e = h * LANES + j
                                ecol = jnp.broadcast_to(
                                    jnp.int32(e), (LANES,))
                                tokt = plsc.load_gather(
                                    tok_v.at[slot], [rows, ecol], mask=m)
                                p0 = splat(p0h[h], jnp.int32(j))
                                plsc.store_scatter(
                                    out_v.at[slot],
                                    [jnp.broadcast_to(jnp.int32(e // 8),
                                                      (LANES,)),
                                     jnp.broadcast_to(jnp.int32(e % 8),
                                                      (LANES,)),
                                     rows],
                                    tokt + p0, mask=m)

        def fire_out(slot, l, sem_o):
            for bb in range(CHUNK // 128):
                pltpu.async_copy(
                    out_v.at[slot, pl.ds(0, embed // 8), pl.ds(0, 8),
                             pl.ds(bb * 128, 128)],
                    out_hbm.at[l, pl.ds(0, embed // 8),
                               wid * (CHUNK // 128) + bb,
                               pl.ds(0, 8), pl.ds(0, 128)], sem_o)

        def phase_fire(slot, l, sem_i, sem_o):
            wait_idx(slot, sem_i)

            @pl.when(l >= 2)
            def _():
                wait_out(slot, sem_o)

            fire_gathers(slot, (sg0, sg1)[slot])

        def phase_drain(slot, l, sem_i, sem_g, sem_o):
            drain_gathers(slot, sem_g)
            transpose_add(slot, l)

            @pl.when(l + 2 < maxlen)
            def _():
                fire_idx(slot, l + 2, sem_i)

            fire_out(slot, l, sem_o)

        fire_idx(0, 0, si0)
        fire_idx(1, 1, si1)

        @pl.loop(0, maxlen, step=2)
        def _(l0):
            phase_fire(0, l0, si0, so0)
            phase_fire(1, l0 + 1, si1, so1)
            phase_drain(0, l0, si0, sg0, so0)
            phase_drain(1, l0 + 1, si1, sg1, so1)

        wait_out(0, so0)
        wait_out(1, so1)

    return sc_kernel(x3, token_table, pos_table)


def kernel(x, token_table, pos_table):
    batch, maxlen = x.shape
    vocab, embed = token_table.shape
    x3 = x.T.reshape(maxlen, batch // GW, GW).astype(jnp.int32)
    out5 = _sc_embed(x3, token_table, pos_table,
                     batch=batch, embed=embed, maxlen=maxlen)
    return out5.transpose(2, 4, 0, 1, 3).reshape(batch, maxlen, embed)


# speedup vs baseline: 3.9832x; 1.0100x over previous
import functools

import jax
import jax.numpy as jnp
from jax import lax
from jax.experimental import pallas as pl
from jax.experimental.pallas import tpu as pltpu
from jax.experimental.pallas import tpu_sc as plsc

NC, NS = 2, 16
NW = NC * NS
GW = 128
NG = 4
CHUNK = GW * NG
LANES = 16


def _sc_embed(x3, token_table, pos_table, *, batch, embed, maxlen):
    mesh = plsc.VectorSubcoreMesh(core_axis_name="c", subcore_axis_name="s")
    egroups = CHUNK // LANES

    @functools.partial(
        pl.kernel,
        out_type=jax.ShapeDtypeStruct(
            (maxlen, embed // 8, batch // 128, 8, 128), jnp.float32),
        mesh=mesh,
        scratch_types=[
            pltpu.VMEM((2, NG, GW), jnp.int32),
            pltpu.VMEM((2, CHUNK, embed), jnp.float32),
            pltpu.VMEM((2, embed // 8, 8, CHUNK + 1), jnp.float32),
            pltpu.VMEM((maxlen + 1, embed), jnp.float32),
            pltpu.SemaphoreType.DMA,
            pltpu.SemaphoreType.DMA,
            pltpu.SemaphoreType.DMA,
            pltpu.SemaphoreType.DMA,
            pltpu.SemaphoreType.DMA,
            pltpu.SemaphoreType.DMA,
            pltpu.SemaphoreType.DMA,
        ],
        compiler_params=pltpu.CompilerParams(use_tc_tiling_on_sc=False,
                                             needs_layout_passes=False),
    )
    def sc_kernel(x_hbm, tok_hbm, pos_hbm, out_hbm,
                  idx_v, tok_v, out_v, pos_v,
                  sp, si0, si1, sg0, sg1, so0, so1):
        wid = lax.axis_index("s") * NC + lax.axis_index("c")
        b0 = wid * CHUNK
        brow = wid * NG

        pltpu.async_copy(pos_hbm, pos_v, sp).wait()

        def fire_idx(slot, l, sem):
            pltpu.async_copy(x_hbm.at[l, pl.ds(brow, NG)], idx_v.at[slot], sem)

        def wait_idx(slot, sem):
            pltpu.make_async_copy(
                x_hbm.at[0, pl.ds(0, NG)], idx_v.at[slot], sem).wait()

        def wait_out(slot, sem):
            for bb in range(CHUNK // 128):
                pltpu.make_async_copy(
                    out_v.at[slot, pl.ds(0, embed // 8), pl.ds(0, 8),
                             pl.ds(bb * 128, 128)],
                    out_hbm.at[0, pl.ds(0, embed // 8), 0,
                               pl.ds(0, 8), pl.ds(0, 128)], sem).wait()

        def fire_gathers(slot, sem_g):
            for k in range(NG):
                pltpu.async_copy(tok_hbm.at[idx_v.at[slot, k]],
                                 tok_v.at[slot, pl.ds(k * GW, GW)], sem_g)

        def drain_gathers(slot, sem_g):
            for k in range(NG):
                pltpu.make_async_copy(
                    tok_hbm.at[pl.ds(0, GW)],
                    tok_v.at[slot, pl.ds(k * GW, GW)], sem_g).wait()

        def splat(vec, lane):
            idx = jnp.broadcast_to(lane.astype(jnp.int32), (LANES,))
            return lax.gather(
                vec, idx[:, None],
                dimension_numbers=lax.GatherDimensionNumbers(
                    offset_dims=(), collapsed_slice_dims=(0,),
                    start_index_map=(0,)),
                slice_sizes=(1,),
                mode=lax.GatherScatterMode.PROMISE_IN_BOUNDS)

        def transpose_add(slot, l):
            iota32 = lax.iota(jnp.int32, LANES)
            p1h = [pos_v[l + 1, pl.ds(h * LANES, LANES)]
                   for h in range(embed // LANES)]
            e8cols = [(h * LANES + iota32) // 8 for h in range(embed // LANES)]
            eincols = [(h * LANES + iota32) % 8 for h in range(embed // LANES)]

            @plsc.parallel_loop(0, CHUNK, unroll=8)
            def _(r):
                rv = jnp.broadcast_to(r, (LANES,))
                for h in range(embed // LANES):
                    tok = tok_v[slot, r, pl.ds(h * LANES, LANES)]
                    plsc.store_scatter(out_v.at[slot],
                                       [e8cols[h], eincols[h], rv],
                                       tok + p1h[h])

            mn = idx_v[slot, 0, pl.ds(0, LANES)]
            for k in range(NG):
                for j in range(GW // LANES):
                    if k == 0 and j == 0:
                        continue
                    mn = jnp.minimum(mn, idx_v[slot, k, pl.ds(j * LANES, LANES)])
            has_zero = lax.reduce_min(mn, (0,)) == 0

            @pl.when(has_zero)
            def _():
                p0h = [pos_v[0, pl.ds(h * LANES, LANES)]
                       for h in range(embed // LANES)]

                @pl.loop(0, egroups)
                def _(g):
                    xg = idx_v[slot, g // (GW // LANES),
                               pl.ds((g % (GW // LANES)) * LANES, LANES)]
                    m = xg == 0
                    rows = g * LANES + iota32

                    @pl.when(jnp.any(m))
                    def _():
                        for h in range(embed // LANES):
                            for j in range(LANES):
                                e = h * LANES + j
                                ecol = jnp.broadcast_to(
                                    jnp.int32(e), (LANES,))
                                tokt = plsc.load_gather(
                                    tok_v.at[slot], [rows, ecol], mask=m)
                                p0 = splat(p0h[h], jnp.int32(j))
                                plsc.store_scatter(
                                    out_v.at[slot],
                                    [jnp.broadcast_to(jnp.int32(e // 8),
                                                      (LANES,)),
                                     jnp.broadcast_to(jnp.int32(e % 8),
                                                      (LANES,)),
                                     rows],
                                    tokt + p0, mask=m)

        def fire_out(slot, l, sem_o):
            for bb in range(CHUNK // 128):
                pltpu.async_copy(
                    out_v.at[slot, pl.ds(0, embed // 8), pl.ds(0, 8),
                             pl.ds(bb * 128, 128)],
                    out_hbm.at[l, pl.ds(0, embed // 8),
                               wid * (CHUNK // 128) + bb,
                               pl.ds(0, 8), pl.ds(0, 128)], sem_o)

        def phase_fire(slot, l, sem_i, sem_o):
            wait_idx(slot, sem_i)
            for k in range(NG):
                for j in range(GW // LANES):
                    sl = (slot, k, pl.ds(j * LANES, LANES))
                    idx_v[sl] = idx_v[sl] * 4

            @pl.when(l >= 2)
            def _():
                wait_out(slot, sem_o)

            fire_gathers(slot, (sg0, sg1)[slot])

        def phase_drain(slot, l, sem_i, sem_g, sem_o):
            drain_gathers(slot, sem_g)
            transpose_add(slot, l)

            @pl.when(l + 2 < maxlen)
            def _():
                fire_idx(slot, l + 2, sem_i)

            fire_out(slot, l, sem_o)

        fire_idx(0, 0, si0)
        fire_idx(1, 1, si1)

        @pl.loop(0, maxlen, step=2)
        def _(l0):
            phase_fire(0, l0, si0, so0)
            phase_fire(1, l0 + 1, si1, so1)
            phase_drain(0, l0, si0, sg0, so0)
            phase_drain(1, l0 + 1, si1, sg1, so1)

        wait_out(0, so0)
        wait_out(1, so1)

    return sc_kernel(x3, token_table, pos_table)


def kernel(x, token_table, pos_table):
    batch, maxlen = x.shape
    vocab, embed = token_table.shape
    x3 = x.T.reshape(maxlen, batch // GW, GW).astype(jnp.int32)
    tok4 = jnp.pad(token_table, ((0, 0), (0, 128 - embed))).reshape(
        4 * vocab, embed)
    out5 = _sc_embed(x3, tok4, pos_table,
                     batch=batch, embed=embed, maxlen=maxlen)
    return out5.transpose(2, 4, 0, 1, 3).reshape(batch, maxlen, embed)
